# Initial kernel scaffold; baseline (speedup 1.0000x reference)
import functools

import jax
import jax.numpy as jnp
import numpy as np
from jax import lax
from jax.experimental import pallas as pl
from jax.experimental.pallas import tpu as pltpu
from jax.experimental.pallas import tpu_sc as plsc

N = 10000
E = 320000
H = 128
ROWW = 144

_NC = 2
_NS = 16
_NW = _NC * _NS
_EPW = E // _NW
_CHUNK = 80
_NCHUNK = _EPW // _CHUNK
_RPS = N // _NS

_SEL_SC_C = np.zeros((16, 48), np.float32)
_SEL_SC_R = np.zeros((16, 48), np.float32)
_SEL_U_C = np.zeros((16, 48), np.float32)
_SEL_U_V = np.zeros((9, 48), np.float32)
for _j in range(3):
    for _i in range(3):
        for _k in range(3):
            _m = 16 * _j + 3 * _i + _k
            _SEL_SC_C[3 * _j + _i, _m] = 1.0
            _SEL_SC_R[3 * _j + _k, _m] = 1.0
            _SEL_U_C[3 * _i + _j, _m] = 1.0
            _SEL_U_V[3 * _j + _k, _m] = 1.0


def _dot(a, b):
    return lax.dot_general(a, b, (((1,), (0,)), ((), ())),
                           preferred_element_type=jnp.float32)


def _silu(x):
    return x * jax.nn.sigmoid(x)


def _sc_gather(hz, row, col):
    mesh = plsc.VectorSubcoreMesh(core_axis_name="c", subcore_axis_name="s")

    @functools.partial(
        pl.kernel,
        out_type=(jax.ShapeDtypeStruct((E, ROWW), jnp.float32),
                  jax.ShapeDtypeStruct((E, ROWW), jnp.float32)),
        mesh=mesh,
        scratch_types=[
            pltpu.VMEM((1, _CHUNK), jnp.int32),
            pltpu.VMEM((1, _CHUNK), jnp.int32),
            pltpu.VMEM((_CHUNK, ROWW), jnp.float32),
            pltpu.VMEM((_CHUNK, ROWW), jnp.float32),
            pltpu.SemaphoreType.DMA,
            pltpu.SemaphoreType.DMA,
        ],
    )
    def k(hz_h, row_h, col_h, outr_h, outc_h, idxr, idxc, bufr, bufc, s1, s2):
        wid = lax.axis_index("s") * _NC + lax.axis_index("c")
        base = wid * _EPW

        def body(j, carry):
            st = pl.multiple_of(base + j * _CHUNK, 8)
            pltpu.sync_copy(row_h.at[pl.ds(st, _CHUNK)], idxr.at[0])
            pltpu.sync_copy(col_h.at[pl.ds(st, _CHUNK)], idxc.at[0])
            cr = pltpu.async_copy(hz_h.at[idxr.at[0]], bufr, s1)
            cc = pltpu.async_copy(hz_h.at[idxc.at[0]], bufc, s2)
            cr.wait()
            cc.wait()
            pltpu.sync_copy(bufr, outr_h.at[pl.ds(st, _CHUNK)])
            pltpu.sync_copy(bufc, outc_h.at[pl.ds(st, _CHUNK)])
            return carry

        lax.fori_loop(0, _NCHUNK, body, 0)

    return k(hz, row, col)


def _sc_scatter(msg, vec, row, zm, zv):
    mesh = plsc.VectorSubcoreMesh(core_axis_name="c", subcore_axis_name="s")

    @functools.partial(
        pl.kernel,
        out_type=(jax.ShapeDtypeStruct((2 * N, H), jnp.float32),
                  jax.ShapeDtypeStruct((2 * N, 16), jnp.float32)),
        mesh=mesh,
        scratch_types=[
            pltpu.VMEM((1, _CHUNK), jnp.int32),
            pltpu.VMEM((_CHUNK, H), jnp.float32),
            pltpu.VMEM((_CHUNK, 16), jnp.float32),
            pltpu.VMEM((_RPS, H), jnp.float32),
            pltpu.VMEM((_RPS, 16), jnp.float32),
            pltpu.VMEM_SHARED((N, H), jnp.float32),
            pltpu.VMEM_SHARED((N, 16), jnp.float32),
        ],
    )
    def k(msg_h, vec_h, row_h, zm_h, zv_h, outm_h, outv_h,
          idx, mb, vb, zmb, zvb, accm, accv):
        cid = lax.axis_index("c")
        sid = lax.axis_index("s")
        wid = sid * _NC + cid
        pltpu.sync_copy(zm_h, zmb)
        pltpu.sync_copy(zv_h, zvb)
        pltpu.sync_copy(zmb, accm.at[pl.ds(sid * _RPS, _RPS)])
        pltpu.sync_copy(zvb, accv.at[pl.ds(sid * _RPS, _RPS)])
        plsc.subcore_barrier()

        base = wid * _EPW

        def body(j, carry):
            st = pl.multiple_of(base + j * _CHUNK, 8)
            pltpu.sync_copy(row_h.at[pl.ds(st, _CHUNK)], idx.at[0])
            pltpu.sync_copy(msg_h.at[pl.ds(st, _CHUNK)], mb)
            pltpu.sync_copy(vec_h.at[pl.ds(st, _CHUNK)], vb)
            pltpu.sync_copy(mb, accm.at[idx.at[0]], add=True)
            pltpu.sync_copy(vb, accv.at[idx.at[0]], add=True)
            return carry

        lax.fori_loop(0, _NCHUNK, body, 0)
        plsc.subcore_barrier()
        pltpu.sync_copy(accm.at[pl.ds(sid * _RPS, _RPS)], zmb)
        pltpu.sync_copy(zmb, outm_h.at[pl.ds(cid * N + sid * _RPS, _RPS)])
        pltpu.sync_copy(accv.at[pl.ds(sid * _RPS, _RPS)], zvb)
        pltpu.sync_copy(zvb, outv_h.at[pl.ds(cid * N + sid * _RPS, _RPS)])

    return k(msg, vec, row, zm, zv)


def _edge_body(hzr, hzc, ef, w1s, w1r, w1c, w1e, b1, w2, b2,
               ow1, ob1, w2v, b2v, scc, scr, ucc, msg_out, vec_out):
    hr = hzr[:, :H]
    zr = hzr[:, H:H + 16]
    hc = hzc[:, :H]
    zc = hzc[:, H:H + 16]

    prod = _dot(zc, scc[...]) * _dot(zr, scr[...])
    scal = prod[:, :16] + prod[:, 16:32] + prod[:, 32:48]
    nrm = jnp.sqrt(jnp.sum(scal * scal, axis=1, keepdims=True))
    scal = scal / jnp.maximum(nrm, 1e-12)

    acc = (_dot(scal, w1s[...]) + _dot(hr, w1r[...]) + _dot(hc, w1c[...])
           + _dot(ef[...], w1e[...]) + b1[...])
    x = _silu(acc)
    m = _silu(_dot(x, w2[...]) + b2[...])
    msg_out[...] = m

    y = _silu(_dot(m, ow1[...]) + ob1[...])
    vsel = _dot(y, w2v[...]) + b2v[...]
    p2 = _dot(zc, ucc[...]) * vsel
    vec = p2[:, :16] + p2[:, 16:32] + p2[:, 32:48]
    cnt = (lax.broadcasted_iota(jnp.int32, vec.shape, 1) == 9)
    vec_out[...] = vec + cnt.astype(jnp.float32)


def _node_body(am0, am1, av0, av1, hb, vb, nw1, nb1, nw2, nb2,
               hout, vout):
    tot = am0[...] + am1[...]
    vt = av0[...] + av1[...]
    cnt = jnp.maximum(vt[:, 9:10], 1.0)
    vout[...] = vt / cnt + vb[...]
    hbv = hb[...]
    z = _silu(_dot(hbv, nw1[:H]) + _dot(tot, nw1[H:]) + nb1[...])
    hout[...] = _dot(z, nw2[...]) + nb2[...] + hbv


_BE = 1280
_BN = 1250


def _tc_edge(hzr, hzc, ef, *ws):
    grid = E // _BE
    full = lambda a: pl.BlockSpec(a.shape, lambda i: (0,) * a.ndim)
    in_specs = [
        pl.BlockSpec((_BE, ROWW), lambda i: (i, 0)),
        pl.BlockSpec((_BE, ROWW), lambda i: (i, 0)),
        pl.BlockSpec((_BE, 4), lambda i: (i, 0)),
    ] + [full(w) for w in ws]
    return pl.pallas_call(
        _edge_body,
        grid=grid,
        in_specs=in_specs,
        out_specs=(pl.BlockSpec((_BE, H), lambda i: (i, 0)),
                   pl.BlockSpec((_BE, 16), lambda i: (i, 0))),
        out_shape=(jax.ShapeDtypeStruct((E, H), jnp.float32),
                   jax.ShapeDtypeStruct((E, 16), jnp.float32)),
        compiler_params=pltpu.CompilerParams(
            dimension_semantics=("arbitrary",)),
    )(hzr, hzc, ef, *ws)


def _tc_node(accm, accv, h, vb, nw1, nb1, nw2, nb2):
    grid = N // _BN
    full = lambda a: pl.BlockSpec(a.shape, lambda i: (0,) * a.ndim)
    in_specs = [
        pl.BlockSpec((_BN, H), lambda i: (i, 0)),
        pl.BlockSpec((_BN, H), lambda i: (i + grid, 0)),
        pl.BlockSpec((_BN, 16), lambda i: (i, 0)),
        pl.BlockSpec((_BN, 16), lambda i: (i + grid, 0)),
        pl.BlockSpec((_BN, H), lambda i: (i, 0)),
        pl.BlockSpec((_BN, 16), lambda i: (i, 0)),
        full(nw1), full(nb1), full(nw2), full(nb2),
    ]
    return pl.pallas_call(
        _node_body,
        grid=grid,
        in_specs=in_specs,
        out_specs=(pl.BlockSpec((_BN, H), lambda i: (i, 0)),
                   pl.BlockSpec((_BN, 16), lambda i: (i, 0))),
        out_shape=(jax.ShapeDtypeStruct((N, H), jnp.float32),
                   jax.ShapeDtypeStruct((N, 16), jnp.float32)),
        compiler_params=pltpu.CompilerParams(
            dimension_semantics=("arbitrary",)),
    )(accm, accm, accv, accv, h, vb, nw1, nb1, nw2, nb2)


def kernel(vectors, h, edge_index, edge_fea,
           in_w1, in_b1, in_w2, in_b2,
           out_w1, out_b1, out_w2, out_b2,
           node_w1, node_b1, node_w2, node_b2):
    row = edge_index[0].astype(jnp.int32)
    col = edge_index[1].astype(jnp.int32)
    vec9 = vectors.reshape(N, 9)
    hz = jnp.concatenate(
        [h, vec9, jnp.zeros((N, ROWW - H - 9), jnp.float32)], axis=1)
    vb16 = jnp.concatenate([vec9, jnp.zeros((N, 7), jnp.float32)], axis=1)

    hzr, hzc = _sc_gather(hz, row, col)

    scc = jnp.asarray(_SEL_SC_C)
    scr = jnp.asarray(_SEL_SC_R)
    ucc = jnp.asarray(_SEL_U_C)
    w2v = out_w2 @ jnp.asarray(_SEL_U_V)
    b2v = out_b2.reshape(1, 9) @ jnp.asarray(_SEL_U_V)
    w1s = jnp.concatenate([in_w1[:9], jnp.zeros((7, H), jnp.float32)], axis=0)
    ws = (w1s, in_w1[9:9 + H], in_w1[9 + H:9 + 2 * H], in_w1[9 + 2 * H:],
          in_b1.reshape(1, H), in_w2, in_b2.reshape(1, H),
          out_w1, out_b1.reshape(1, H), w2v, b2v, scc, scr, ucc)

    msg, vec = _tc_edge(hzr, hzc, edge_fea, *ws)

    zm = jnp.zeros((_RPS, H), jnp.float32)
    zv = jnp.zeros((_RPS, 16), jnp.float32)
    accm, accv = _sc_scatter(msg, vec, row, zm, zv)

    h_new, vout16 = _tc_node(accm, accv, h, vb16,
                             node_w1, node_b1.reshape(1, H),
                             node_w2, node_b2.reshape(1, H))
    vout = vout16[:, :9].reshape(N, 3, 3)
    return (vout, h_new)


# speedup vs baseline: 3.6150x; 3.6150x over previous
import functools

import jax
import jax.numpy as jnp
import numpy as np
from jax import lax
from jax.experimental import pallas as pl
from jax.experimental.pallas import tpu as pltpu
from jax.experimental.pallas import tpu_sc as plsc

N = 10000
E = 320000
H = 128
ROWW = 144

_NC = 2
_NS = 16
_NW = _NC * _NS
_EPW = E // _NW
_CHUNK = 80
_NCHUNK = _EPW // _CHUNK
_RPS = N // _NS

_SEL_SC_C = np.zeros((16, 48), np.float32)
_SEL_SC_R = np.zeros((16, 48), np.float32)
_SEL_U_C = np.zeros((16, 48), np.float32)
_SEL_U_V = np.zeros((9, 48), np.float32)
for _j in range(3):
    for _i in range(3):
        for _k in range(3):
            _m = 16 * _j + 3 * _i + _k
            _SEL_SC_C[3 * _j + _i, _m] = 1.0
            _SEL_SC_R[3 * _j + _k, _m] = 1.0
            _SEL_U_C[3 * _i + _j, _m] = 1.0
            _SEL_U_V[3 * _j + _k, _m] = 1.0


def _dot(a, b):
    return lax.dot_general(a, b, (((1,), (0,)), ((), ())),
                           preferred_element_type=jnp.float32)


def _silu(x):
    return x * jax.nn.sigmoid(x)


def _sc_gather(hz, row, col):
    mesh = plsc.VectorSubcoreMesh(core_axis_name="c", subcore_axis_name="s")

    @functools.partial(
        pl.kernel,
        out_type=(jax.ShapeDtypeStruct((E, ROWW), jnp.float32),
                  jax.ShapeDtypeStruct((E, ROWW), jnp.float32)),
        mesh=mesh,
        scratch_types=[
            pltpu.VMEM((1, _CHUNK), jnp.int32),
            pltpu.VMEM((1, _CHUNK), jnp.int32),
            pltpu.VMEM((_CHUNK, ROWW), jnp.float32),
            pltpu.VMEM((_CHUNK, ROWW), jnp.float32),
            pltpu.SemaphoreType.DMA,
            pltpu.SemaphoreType.DMA,
        ],
        compiler_params=pltpu.CompilerParams(use_tc_tiling_on_sc=False),
    )
    def k(hz_h, row_h, col_h, outr_h, outc_h, idxr, idxc, bufr, bufc, s1, s2):
        wid = lax.axis_index("s") * _NC + lax.axis_index("c")
        base = wid * _EPW

        def body(j, carry):
            st = pl.multiple_of(base + j * _CHUNK, 8)
            pltpu.sync_copy(row_h.at[pl.ds(st, _CHUNK)], idxr.at[0])
            pltpu.sync_copy(col_h.at[pl.ds(st, _CHUNK)], idxc.at[0])
            cr = pltpu.async_copy(hz_h.at[idxr.at[0]], bufr, s1)
            cc = pltpu.async_copy(hz_h.at[idxc.at[0]], bufc, s2)
            cr.wait()
            cc.wait()
            pltpu.sync_copy(bufr, outr_h.at[pl.ds(st, _CHUNK)])
            pltpu.sync_copy(bufc, outc_h.at[pl.ds(st, _CHUNK)])
            return carry

        lax.fori_loop(0, _NCHUNK, body, 0)

    return k(hz, row, col)


_HC = H // 2
_EPS = E // _NS
_NCHUNK_SC = _EPS // _CHUNK


def _sc_scatter(msg, vec, row, zm, zv):
    mesh = plsc.VectorSubcoreMesh(core_axis_name="c", subcore_axis_name="s")

    @functools.partial(
        pl.kernel,
        out_type=(jax.ShapeDtypeStruct((N, H), jnp.float32),
                  jax.ShapeDtypeStruct((N, 16), jnp.float32)),
        mesh=mesh,
        scratch_types=[
            pltpu.VMEM((1, _CHUNK), jnp.int32),
            pltpu.VMEM((_CHUNK, _HC), jnp.float32),
            pltpu.VMEM((_CHUNK, 16), jnp.float32),
            pltpu.VMEM((_RPS, _HC), jnp.float32),
            pltpu.VMEM((_RPS, 16), jnp.float32),
            pltpu.VMEM_SHARED((N, _HC), jnp.float32),
            pltpu.VMEM_SHARED((N, 16), jnp.float32),
        ],
        compiler_params=pltpu.CompilerParams(use_tc_tiling_on_sc=False),
    )
    def k(msg_h, vec_h, row_h, zm_h, zv_h, outm_h, outv_h,
          idx, mb, vb, zmb, zvb, accm, accv):
        cid = lax.axis_index("c")
        sid = lax.axis_index("s")
        pltpu.sync_copy(zm_h, zmb)
        pltpu.sync_copy(zmb, accm.at[pl.ds(sid * _RPS, _RPS)])

        @pl.when(cid == 0)
        def _():
            pltpu.sync_copy(zv_h, zvb)
            pltpu.sync_copy(zvb, accv.at[pl.ds(sid * _RPS, _RPS)])

        plsc.subcore_barrier()

        base = sid * _EPS
        coff = cid * _HC

        def body(j, carry):
            st = pl.multiple_of(base + j * _CHUNK, 8)
            pltpu.sync_copy(row_h.at[pl.ds(st, _CHUNK)], idx.at[0])
            pltpu.sync_copy(msg_h.at[pl.ds(st, _CHUNK), pl.ds(coff, _HC)], mb)
            pltpu.sync_copy(mb, accm.at[idx.at[0]], add=True)

            @pl.when(cid == 0)
            def _():
                pltpu.sync_copy(vec_h.at[pl.ds(st, _CHUNK)], vb)
                pltpu.sync_copy(vb, accv.at[idx.at[0]], add=True)

            return carry

        lax.fori_loop(0, _NCHUNK_SC, body, 0)
        plsc.subcore_barrier()
        pltpu.sync_copy(accm.at[pl.ds(sid * _RPS, _RPS)], zmb)
        pltpu.sync_copy(zmb, outm_h.at[pl.ds(sid * _RPS, _RPS), pl.ds(coff, _HC)])

        @pl.when(cid == 0)
        def _():
            pltpu.sync_copy(accv.at[pl.ds(sid * _RPS, _RPS)], zvb)
            pltpu.sync_copy(zvb, outv_h.at[pl.ds(sid * _RPS, _RPS)])

    return k(msg, vec, row, zm, zv)


def _edge_body(hzr, hzc, ef, w1s, w1r, w1c, w1e, b1, w2, b2,
               ow1, ob1, w2v, b2v, scc, scr, ucc, msg_out, vec_out):
    hr = hzr[:, :H]
    zr = hzr[:, H:H + 16]
    hc = hzc[:, :H]
    zc = hzc[:, H:H + 16]

    prod = _dot(zc, scc[...]) * _dot(zr, scr[...])
    scal = prod[:, :16] + prod[:, 16:32] + prod[:, 32:48]
    nrm = jnp.sqrt(jnp.sum(scal * scal, axis=1, keepdims=True))
    scal = scal / jnp.maximum(nrm, 1e-12)

    acc = (_dot(scal, w1s[...]) + _dot(hr, w1r[...]) + _dot(hc, w1c[...])
           + _dot(ef[...], w1e[...]) + b1[...])
    x = _silu(acc)
    m = _silu(_dot(x, w2[...]) + b2[...])
    msg_out[...] = m

    y = _silu(_dot(m, ow1[...]) + ob1[...])
    vsel = _dot(y, w2v[...]) + b2v[...]
    p2 = _dot(zc, ucc[...]) * vsel
    vec = p2[:, :16] + p2[:, 16:32] + p2[:, 32:48]
    cnt = (lax.broadcasted_iota(jnp.int32, vec.shape, 1) == 9)
    vec_out[...] = vec + cnt.astype(jnp.float32)


def _node_body(am, av, hb, vb, nw1, nb1, nw2, nb2,
               hout, vout):
    tot = am[...]
    vt = av[...]
    cnt = jnp.maximum(vt[:, 9:10], 1.0)
    vout[...] = vt / cnt + vb[...]
    hbv = hb[...]
    z = _silu(_dot(hbv, nw1[:H]) + _dot(tot, nw1[H:]) + nb1[...])
    hout[...] = _dot(z, nw2[...]) + nb2[...] + hbv


_BE = 1280
_BN = 1000


def _tc_edge(hzr, hzc, ef, *ws):
    grid = E // _BE
    full = lambda a: pl.BlockSpec(a.shape, lambda i: (0,) * a.ndim)
    in_specs = [
        pl.BlockSpec((_BE, ROWW), lambda i: (i, 0)),
        pl.BlockSpec((_BE, ROWW), lambda i: (i, 0)),
        pl.BlockSpec((_BE, 4), lambda i: (i, 0)),
    ] + [full(w) for w in ws]
    return pl.pallas_call(
        _edge_body,
        grid=grid,
        in_specs=in_specs,
        out_specs=(pl.BlockSpec((_BE, H), lambda i: (i, 0)),
                   pl.BlockSpec((_BE, 16), lambda i: (i, 0))),
        out_shape=(jax.ShapeDtypeStruct((E, H), jnp.float32),
                   jax.ShapeDtypeStruct((E, 16), jnp.float32)),
        compiler_params=pltpu.CompilerParams(
            dimension_semantics=("arbitrary",)),
    )(hzr, hzc, ef, *ws)


def _tc_node(accm, accv, h, vb, nw1, nb1, nw2, nb2):
    grid = N // _BN
    full = lambda a: pl.BlockSpec(a.shape, lambda i: (0,) * a.ndim)
    in_specs = [
        pl.BlockSpec((_BN, H), lambda i: (i, 0)),
        pl.BlockSpec((_BN, 16), lambda i: (i, 0)),
        pl.BlockSpec((_BN, H), lambda i: (i, 0)),
        pl.BlockSpec((_BN, 16), lambda i: (i, 0)),
        full(nw1), full(nb1), full(nw2), full(nb2),
    ]
    return pl.pallas_call(
        _node_body,
        grid=grid,
        in_specs=in_specs,
        out_specs=(pl.BlockSpec((_BN, H), lambda i: (i, 0)),
                   pl.BlockSpec((_BN, 16), lambda i: (i, 0))),
        out_shape=(jax.ShapeDtypeStruct((N, H), jnp.float32),
                   jax.ShapeDtypeStruct((N, 16), jnp.float32)),
        compiler_params=pltpu.CompilerParams(
            dimension_semantics=("arbitrary",)),
    )(accm, accv, h, vb, nw1, nb1, nw2, nb2)


def kernel(vectors, h, edge_index, edge_fea,
           in_w1, in_b1, in_w2, in_b2,
           out_w1, out_b1, out_w2, out_b2,
           node_w1, node_b1, node_w2, node_b2):
    row = edge_index[0].astype(jnp.int32)
    col = edge_index[1].astype(jnp.int32)
    vec9 = vectors.reshape(N, 9)
    hz = jnp.concatenate(
        [h, vec9, jnp.zeros((N, ROWW - H - 9), jnp.float32)], axis=1)
    vb16 = jnp.concatenate([vec9, jnp.zeros((N, 7), jnp.float32)], axis=1)

    hzr, hzc = _sc_gather(hz, row, col)

    scc = jnp.asarray(_SEL_SC_C)
    scr = jnp.asarray(_SEL_SC_R)
    ucc = jnp.asarray(_SEL_U_C)
    w2v = out_w2 @ jnp.asarray(_SEL_U_V)
    b2v = out_b2.reshape(1, 9) @ jnp.asarray(_SEL_U_V)
    w1s = jnp.concatenate([in_w1[:9], jnp.zeros((7, H), jnp.float32)], axis=0)
    ws = (w1s, in_w1[9:9 + H], in_w1[9 + H:9 + 2 * H], in_w1[9 + 2 * H:],
          in_b1.reshape(1, H), in_w2, in_b2.reshape(1, H),
          out_w1, out_b1.reshape(1, H), w2v, b2v, scc, scr, ucc)

    msg, vec = _tc_edge(hzr, hzc, edge_fea, *ws)

    zm = jnp.zeros((_RPS, _HC), jnp.float32)
    zv = jnp.zeros((_RPS, 16), jnp.float32)
    accm, accv = _sc_scatter(msg, vec, row, zm, zv)

    h_new, vout16 = _tc_node(accm, accv, h, vb16,
                             node_w1, node_b1.reshape(1, H),
                             node_w2, node_b2.reshape(1, H))
    vout = vout16[:, :9].reshape(N, 3, 3)
    return (vout, h_new)


# speedup vs baseline: 4.1851x; 1.1577x over previous
import functools

import jax
import jax.numpy as jnp
import numpy as np
from jax import lax
from jax.experimental import pallas as pl
from jax.experimental.pallas import tpu as pltpu
from jax.experimental.pallas import tpu_sc as plsc

N = 10000
E = 320000
H = 128
ROWW = 144

_NC = 2
_NS = 16
_NW = _NC * _NS
_EPW = E // _NW
_CHUNK = 80
_NCHUNK = _EPW // _CHUNK
_RPS = N // _NS

_SEL_SC_C = np.zeros((16, 48), np.float32)
_SEL_SC_R = np.zeros((16, 48), np.float32)
_SEL_U_C = np.zeros((16, 48), np.float32)
_SEL_U_V = np.zeros((9, 48), np.float32)
for _j in range(3):
    for _i in range(3):
        for _k in range(3):
            _m = 16 * _j + 3 * _i + _k
            _SEL_SC_C[3 * _j + _i, _m] = 1.0
            _SEL_SC_R[3 * _j + _k, _m] = 1.0
            _SEL_U_C[3 * _i + _j, _m] = 1.0
            _SEL_U_V[3 * _j + _k, _m] = 1.0


def _dot(a, b):
    return lax.dot_general(a, b, (((1,), (0,)), ((), ())),
                           preferred_element_type=jnp.float32)


def _silu(x):
    return x * jax.nn.sigmoid(x)


def _sc_gather_h(h, row2, col2):
    mesh = plsc.VectorSubcoreMesh(core_axis_name="c", subcore_axis_name="s")

    @functools.partial(
        pl.kernel,
        out_type=(jax.ShapeDtypeStruct((E, H), jnp.float32),
                  jax.ShapeDtypeStruct((E, H), jnp.float32)),
        mesh=mesh,
        scratch_types=[
            pltpu.VMEM((1, _CHUNK), jnp.int32),
            pltpu.VMEM((1, _CHUNK), jnp.int32),
            pltpu.VMEM((_CHUNK, H), jnp.float32),
            pltpu.VMEM((_CHUNK, H), jnp.float32),
            pltpu.SemaphoreType.DMA,
            pltpu.SemaphoreType.DMA,
        ],
    )
    def k(h_h, row_h, col_h, outr_h, outc_h, idxr, idxc, bufr, bufc, s1, s2):
        wid = lax.axis_index("s") * _NC + lax.axis_index("c")
        base = wid * _EPW

        def body(j, carry):
            ci = wid * _NCHUNK + j
            st = pl.multiple_of(base + j * _CHUNK, 8)
            pltpu.sync_copy(row_h.at[ci], idxr.at[0])
            pltpu.sync_copy(col_h.at[ci], idxc.at[0])
            cr = pltpu.async_copy(h_h.at[idxr.at[0]], bufr, s1)
            cc = pltpu.async_copy(h_h.at[idxc.at[0]], bufc, s2)
            cr.wait()
            cc.wait()
            pltpu.sync_copy(bufr, outr_h.at[pl.ds(st, _CHUNK)])
            pltpu.sync_copy(bufc, outc_h.at[pl.ds(st, _CHUNK)])
            return carry

        lax.fori_loop(0, _NCHUNK, body, 0)

    return k(h, row2, col2)


def _sc_gather_z(z16, row, col):
    mesh = plsc.VectorSubcoreMesh(core_axis_name="c", subcore_axis_name="s")

    @functools.partial(
        pl.kernel,
        out_type=(jax.ShapeDtypeStruct((E, 16), jnp.float32),
                  jax.ShapeDtypeStruct((E, 16), jnp.float32)),
        mesh=mesh,
        scratch_types=[
            pltpu.VMEM((1, _CHUNK), jnp.int32),
            pltpu.VMEM((1, _CHUNK), jnp.int32),
            pltpu.VMEM((_CHUNK, 16), jnp.float32),
            pltpu.VMEM((_CHUNK, 16), jnp.float32),
            pltpu.SemaphoreType.DMA,
            pltpu.SemaphoreType.DMA,
        ],
        compiler_params=pltpu.CompilerParams(use_tc_tiling_on_sc=False),
    )
    def k(z_h, row_h, col_h, outr_h, outc_h, idxr, idxc, bufr, bufc, s1, s2):
        wid = lax.axis_index("s") * _NC + lax.axis_index("c")
        base = wid * _EPW

        def body(j, carry):
            st = pl.multiple_of(base + j * _CHUNK, 8)
            pltpu.sync_copy(row_h.at[pl.ds(st, _CHUNK)], idxr.at[0])
            pltpu.sync_copy(col_h.at[pl.ds(st, _CHUNK)], idxc.at[0])
            cr = pltpu.async_copy(z_h.at[idxr.at[0]], bufr, s1)
            cc = pltpu.async_copy(z_h.at[idxc.at[0]], bufc, s2)
            cr.wait()
            cc.wait()
            pltpu.sync_copy(bufr, outr_h.at[pl.ds(st, _CHUNK)])
            pltpu.sync_copy(bufc, outc_h.at[pl.ds(st, _CHUNK)])
            return carry

        lax.fori_loop(0, _NCHUNK, body, 0)

    return k(z16, row, col)


_HC = H // 2
_EPS = E // _NS
_NCHUNK_SC = _EPS // _CHUNK


def _sc_scatter(msg, vec, row, zm, zv):
    mesh = plsc.VectorSubcoreMesh(core_axis_name="c", subcore_axis_name="s")

    @functools.partial(
        pl.kernel,
        out_type=(jax.ShapeDtypeStruct((N, H), jnp.float32),
                  jax.ShapeDtypeStruct((N, 16), jnp.float32)),
        mesh=mesh,
        scratch_types=[
            pltpu.VMEM((1, _CHUNK), jnp.int32),
            pltpu.VMEM((_CHUNK, _HC), jnp.float32),
            pltpu.VMEM((_CHUNK, 16), jnp.float32),
            pltpu.VMEM((_RPS, _HC), jnp.float32),
            pltpu.VMEM((_RPS, 16), jnp.float32),
            pltpu.VMEM_SHARED((N, _HC), jnp.float32),
            pltpu.VMEM_SHARED((N, 16), jnp.float32),
        ],
        compiler_params=pltpu.CompilerParams(use_tc_tiling_on_sc=False),
    )
    def k(msg_h, vec_h, row_h, zm_h, zv_h, outm_h, outv_h,
          idx, mb, vb, zmb, zvb, accm, accv):
        cid = lax.axis_index("c")
        sid = lax.axis_index("s")
        pltpu.sync_copy(zm_h, zmb)
        pltpu.sync_copy(zmb, accm.at[pl.ds(sid * _RPS, _RPS)])

        @pl.when(cid == 0)
        def _():
            pltpu.sync_copy(zv_h, zvb)
            pltpu.sync_copy(zvb, accv.at[pl.ds(sid * _RPS, _RPS)])

        plsc.subcore_barrier()

        base = sid * _EPS
        coff = cid * _HC

        def body(j, carry):
            st = pl.multiple_of(base + j * _CHUNK, 8)
            pltpu.sync_copy(row_h.at[pl.ds(st, _CHUNK)], idx.at[0])
            pltpu.sync_copy(msg_h.at[pl.ds(st, _CHUNK), pl.ds(coff, _HC)], mb)
            pltpu.sync_copy(mb, accm.at[idx.at[0]], add=True)

            @pl.when(cid == 0)
            def _():
                pltpu.sync_copy(vec_h.at[pl.ds(st, _CHUNK)], vb)
                pltpu.sync_copy(vb, accv.at[idx.at[0]], add=True)

            return carry

        lax.fori_loop(0, _NCHUNK_SC, body, 0)
        plsc.subcore_barrier()
        pltpu.sync_copy(accm.at[pl.ds(sid * _RPS, _RPS)], zmb)
        pltpu.sync_copy(zmb, outm_h.at[pl.ds(sid * _RPS, _RPS), pl.ds(coff, _HC)])

        @pl.when(cid == 0)
        def _():
            pltpu.sync_copy(accv.at[pl.ds(sid * _RPS, _RPS)], zvb)
            pltpu.sync_copy(zvb, outv_h.at[pl.ds(sid * _RPS, _RPS)])

    return k(msg, vec, row, zm, zv)


def _edge_body(hr_ref, hc_ref, zr_ref, zc_ref, ef, w1s, w1r, w1c, w1e, b1,
               w2, b2, ow1, ob1, w2v, b2v, scc, scr, ucc, msg_out, vec_out):
    hr = hr_ref[...]
    zr = zr_ref[...]
    hc = hc_ref[...]
    zc = zc_ref[...]

    prod = _dot(zc, scc[...]) * _dot(zr, scr[...])
    scal = prod[:, :16] + prod[:, 16:32] + prod[:, 32:48]
    nrm = jnp.sqrt(jnp.sum(scal * scal, axis=1, keepdims=True))
    scal = scal / jnp.maximum(nrm, 1e-12)

    acc = (_dot(scal, w1s[...]) + _dot(hr, w1r[...]) + _dot(hc, w1c[...])
           + _dot(ef[...], w1e[...]) + b1[...])
    x = _silu(acc)
    m = _silu(_dot(x, w2[...]) + b2[...])
    msg_out[...] = m

    y = _silu(_dot(m, ow1[...]) + ob1[...])
    vsel = _dot(y, w2v[...]) + b2v[...]
    p2 = _dot(zc, ucc[...]) * vsel
    vec = p2[:, :16] + p2[:, 16:32] + p2[:, 32:48]
    cnt = (lax.broadcasted_iota(jnp.int32, vec.shape, 1) == 9)
    vec_out[...] = vec + cnt.astype(jnp.float32)


def _node_body(am, av, hb, vb, nw1, nb1, nw2, nb2,
               hout, vout):
    tot = am[...]
    vt = av[...]
    cnt = jnp.maximum(vt[:, 9:10], 1.0)
    vout[...] = vt / cnt + vb[...]
    hbv = hb[...]
    z = _silu(_dot(hbv, nw1[:H]) + _dot(tot, nw1[H:]) + nb1[...])
    hout[...] = _dot(z, nw2[...]) + nb2[...] + hbv


_BE = 1280
_BN = 1000


def _tc_edge(hr, hc, zr, zc, ef, *ws):
    grid = E // _BE
    full = lambda a: pl.BlockSpec(a.shape, lambda i: (0,) * a.ndim)
    in_specs = [
        pl.BlockSpec((_BE, H), lambda i: (i, 0)),
        pl.BlockSpec((_BE, H), lambda i: (i, 0)),
        pl.BlockSpec((_BE, 16), lambda i: (i, 0)),
        pl.BlockSpec((_BE, 16), lambda i: (i, 0)),
        pl.BlockSpec((_BE, 4), lambda i: (i, 0)),
    ] + [full(w) for w in ws]
    return pl.pallas_call(
        _edge_body,
        grid=grid,
        in_specs=in_specs,
        out_specs=(pl.BlockSpec((_BE, H), lambda i: (i, 0)),
                   pl.BlockSpec((_BE, 16), lambda i: (i, 0))),
        out_shape=(jax.ShapeDtypeStruct((E, H), jnp.float32),
                   jax.ShapeDtypeStruct((E, 16), jnp.float32)),
        compiler_params=pltpu.CompilerParams(
            dimension_semantics=("arbitrary",)),
    )(hr, hc, zr, zc, ef, *ws)


def _tc_node(accm, accv, h, vb, nw1, nb1, nw2, nb2):
    grid = N // _BN
    full = lambda a: pl.BlockSpec(a.shape, lambda i: (0,) * a.ndim)
    in_specs = [
        pl.BlockSpec((_BN, H), lambda i: (i, 0)),
        pl.BlockSpec((_BN, 16), lambda i: (i, 0)),
        pl.BlockSpec((_BN, H), lambda i: (i, 0)),
        pl.BlockSpec((_BN, 16), lambda i: (i, 0)),
        full(nw1), full(nb1), full(nw2), full(nb2),
    ]
    return pl.pallas_call(
        _node_body,
        grid=grid,
        in_specs=in_specs,
        out_specs=(pl.BlockSpec((_BN, H), lambda i: (i, 0)),
                   pl.BlockSpec((_BN, 16), lambda i: (i, 0))),
        out_shape=(jax.ShapeDtypeStruct((N, H), jnp.float32),
                   jax.ShapeDtypeStruct((N, 16), jnp.float32)),
        compiler_params=pltpu.CompilerParams(
            dimension_semantics=("arbitrary",)),
    )(accm, accv, h, vb, nw1, nb1, nw2, nb2)


def kernel(vectors, h, edge_index, edge_fea,
           in_w1, in_b1, in_w2, in_b2,
           out_w1, out_b1, out_w2, out_b2,
           node_w1, node_b1, node_w2, node_b2):
    row = edge_index[0].astype(jnp.int32)
    col = edge_index[1].astype(jnp.int32)
    vec9 = vectors.reshape(N, 9)
    vb16 = jnp.concatenate([vec9, jnp.zeros((N, 7), jnp.float32)], axis=1)

    hr, hc = _sc_gather_h(h, row.reshape(E // _CHUNK, _CHUNK),
                          col.reshape(E // _CHUNK, _CHUNK))
    zr, zc = _sc_gather_z(vb16, row, col)

    scc = jnp.asarray(_SEL_SC_C)
    scr = jnp.asarray(_SEL_SC_R)
    ucc = jnp.asarray(_SEL_U_C)
    w2v = out_w2 @ jnp.asarray(_SEL_U_V)
    b2v = out_b2.reshape(1, 9) @ jnp.asarray(_SEL_U_V)
    w1s = jnp.concatenate([in_w1[:9], jnp.zeros((7, H), jnp.float32)], axis=0)
    ws = (w1s, in_w1[9:9 + H], in_w1[9 + H:9 + 2 * H], in_w1[9 + 2 * H:],
          in_b1.reshape(1, H), in_w2, in_b2.reshape(1, H),
          out_w1, out_b1.reshape(1, H), w2v, b2v, scc, scr, ucc)

    msg, vec = _tc_edge(hr, hc, zr, zc, edge_fea, *ws)

    zm = jnp.zeros((_RPS, _HC), jnp.float32)
    zv = jnp.zeros((_RPS, 16), jnp.float32)
    accm, accv = _sc_scatter(msg, vec, row, zm, zv)

    h_new, vout16 = _tc_node(accm, accv, h, vb16,
                             node_w1, node_b1.reshape(1, H),
                             node_w2, node_b2.reshape(1, H))
    vout = vout16[:, :9].reshape(N, 3, 3)
    return (vout, h_new)


# speedup vs baseline: 5.7149x; 1.3655x over previous
import functools

import jax
import jax.numpy as jnp
import numpy as np
from jax import lax
from jax.experimental import pallas as pl
from jax.experimental.pallas import tpu as pltpu
from jax.experimental.pallas import tpu_sc as plsc

N = 10000
E = 320000
H = 128
ROWW = 144

_NC = 2
_NS = 16
_NW = _NC * _NS
_EPW = E // _NW
_CHUNK = 80
_NCHUNK = _EPW // _CHUNK
_RPS = N // _NS

_SEL_SC_C = np.zeros((16, 48), np.float32)
_SEL_SC_R = np.zeros((16, 48), np.float32)
_SEL_U_C = np.zeros((16, 48), np.float32)
_SEL_U_V = np.zeros((9, 48), np.float32)
for _j in range(3):
    for _i in range(3):
        for _k in range(3):
            _m = 16 * _j + 3 * _i + _k
            _SEL_SC_C[3 * _j + _i, _m] = 1.0
            _SEL_SC_R[3 * _j + _k, _m] = 1.0
            _SEL_U_C[3 * _i + _j, _m] = 1.0
            _SEL_U_V[3 * _j + _k, _m] = 1.0


def _dot(a, b):
    return lax.dot_general(a, b, (((1,), (0,)), ((), ())),
                           preferred_element_type=jnp.float32)


def _silu(x):
    return x * jax.nn.sigmoid(x)


_NBUF = 5
_NOUTER = _NCHUNK // _NBUF


def _make_gather(width, tc_tiling, idx2d):
    mesh = plsc.VectorSubcoreMesh(core_axis_name="c", subcore_axis_name="s")

    @functools.partial(
        pl.kernel,
        out_type=(jax.ShapeDtypeStruct((E, width), jnp.float32),
                  jax.ShapeDtypeStruct((E, width), jnp.float32)),
        mesh=mesh,
        scratch_types=[
            pltpu.VMEM((_NBUF, _CHUNK), jnp.int32),
            pltpu.VMEM((_NBUF, _CHUNK), jnp.int32),
            pltpu.VMEM((_NBUF, _CHUNK, width), jnp.float32),
            pltpu.VMEM((_NBUF, _CHUNK, width), jnp.float32),
            pltpu.SemaphoreType.DMA,
            pltpu.SemaphoreType.DMA,
            pltpu.SemaphoreType.DMA,
        ],
        compiler_params=pltpu.CompilerParams(use_tc_tiling_on_sc=tc_tiling),
    )
    def k(t_h, row_h, col_h, outr_h, outc_h, idxr, idxc, bufr, bufc,
          s_i, s_g, s_w):
        wid = lax.axis_index("s") * _NC + lax.axis_index("c")
        base = wid * _EPW
        cbase = wid * _NCHUNK

        def start_idx(j, b):
            if idx2d:
                pltpu.async_copy(row_h.at[cbase + j], idxr.at[b], s_i)
                pltpu.async_copy(col_h.at[cbase + j], idxc.at[b], s_i)
            else:
                st = pl.multiple_of(base + j * _CHUNK, 8)
                pltpu.async_copy(row_h.at[pl.ds(st, _CHUNK)], idxr.at[b], s_i)
                pltpu.async_copy(col_h.at[pl.ds(st, _CHUNK)], idxc.at[b], s_i)

        def wait_idx(b):
            if idx2d:
                pltpu.make_async_copy(row_h.at[cbase], idxr.at[b], s_i).wait()
                pltpu.make_async_copy(col_h.at[cbase], idxc.at[b], s_i).wait()
            else:
                pltpu.make_async_copy(
                    row_h.at[pl.ds(base, _CHUNK)], idxr.at[b], s_i).wait()
                pltpu.make_async_copy(
                    col_h.at[pl.ds(base, _CHUNK)], idxc.at[b], s_i).wait()

        for b in range(_NBUF):
            start_idx(b, b)

        def body(t, carry):
            j0 = t * _NBUF
            for b in range(_NBUF):
                wait_idx(b)
            for b in range(_NBUF):
                pltpu.async_copy(t_h.at[idxr.at[b]], bufr.at[b], s_g)
                pltpu.async_copy(t_h.at[idxc.at[b]], bufc.at[b], s_g)
            for b in range(_NBUF):
                pltpu.make_async_copy(t_h.at[idxr.at[b]], bufr.at[b], s_g).wait()
                pltpu.make_async_copy(t_h.at[idxc.at[b]], bufc.at[b], s_g).wait()
            for b in range(_NBUF):
                st = pl.multiple_of(base + (j0 + b) * _CHUNK, 8)
                pltpu.async_copy(bufr.at[b], outr_h.at[pl.ds(st, _CHUNK)], s_w)
                pltpu.async_copy(bufc.at[b], outc_h.at[pl.ds(st, _CHUNK)], s_w)

            @pl.when(t < _NOUTER - 1)
            def _():
                for b in range(_NBUF):
                    start_idx(j0 + _NBUF + b, b)

            for b in range(_NBUF):
                st = pl.multiple_of(base + (j0 + b) * _CHUNK, 8)
                pltpu.make_async_copy(
                    bufr.at[b], outr_h.at[pl.ds(st, _CHUNK)], s_w).wait()
                pltpu.make_async_copy(
                    bufc.at[b], outc_h.at[pl.ds(st, _CHUNK)], s_w).wait()
            return carry

        lax.fori_loop(0, _NOUTER, body, 0)

    return k


def _sc_gather_h(h, row2, col2):
    return _make_gather(H, True, True)(h, row2, col2)


def _sc_gather_z(z16, row, col):
    return _make_gather(16, False, False)(z16, row, col)


_HC = H // 2
_EPS = E // _NS
_NCHUNK_SC = _EPS // _CHUNK


def _sc_scatter(msg, vec, row, zm, zv):
    mesh = plsc.VectorSubcoreMesh(core_axis_name="c", subcore_axis_name="s")

    @functools.partial(
        pl.kernel,
        out_type=(jax.ShapeDtypeStruct((N, H), jnp.float32),
                  jax.ShapeDtypeStruct((N, 16), jnp.float32)),
        mesh=mesh,
        scratch_types=[
            pltpu.VMEM((_NBUF, _CHUNK), jnp.int32),
            pltpu.VMEM((_NBUF, _CHUNK, _HC), jnp.float32),
            pltpu.VMEM((_NBUF, _CHUNK, 16), jnp.float32),
            pltpu.VMEM_SHARED((N, _HC), jnp.float32),
            pltpu.VMEM_SHARED((N, 16), jnp.float32),
            pltpu.SemaphoreType.DMA,
            pltpu.SemaphoreType.DMA,
            pltpu.SemaphoreType.DMA,
            pltpu.SemaphoreType.DMA,
        ],
        compiler_params=pltpu.CompilerParams(use_tc_tiling_on_sc=False),
    )
    def k(msg_h, vec_h, row_h, zm_h, zv_h, outm_h, outv_h,
          idx, mb, vb, accm, accv, s_i, s_m, s_v, s_a):
        cid = lax.axis_index("c")
        sid = lax.axis_index("s")
        rchunks = [(0, 80), (80, 80), (160, 80), (240, 80), (320, 80),
                   (400, 80), (480, 80), (560, 65)]
        pltpu.sync_copy(zm_h, mb.at[0])
        pltpu.sync_copy(zv_h, vb.at[0])
        for off, ln in rchunks:
            pltpu.sync_copy(mb.at[0, pl.ds(0, ln)],
                            accm.at[pl.ds(sid * _RPS + off, ln)])

            @pl.when(cid == 0)
            def _():
                pltpu.sync_copy(vb.at[0, pl.ds(0, ln)],
                                accv.at[pl.ds(sid * _RPS + off, ln)])

        plsc.subcore_barrier()

        base = sid * _EPS
        coff = cid * _HC

        def start_loads(j, b):
            st = pl.multiple_of(base + j * _CHUNK, 8)
            pltpu.async_copy(row_h.at[pl.ds(st, _CHUNK)], idx.at[b], s_i)
            pltpu.async_copy(
                msg_h.at[pl.ds(st, _CHUNK), pl.ds(coff, _HC)], mb.at[b], s_m)

            @pl.when(cid == 0)
            def _():
                pltpu.async_copy(vec_h.at[pl.ds(st, _CHUNK)], vb.at[b], s_v)

        def wait_loads(b):
            pltpu.make_async_copy(
                row_h.at[pl.ds(base, _CHUNK)], idx.at[b], s_i).wait()
            pltpu.make_async_copy(
                msg_h.at[pl.ds(base, _CHUNK), pl.ds(coff, _HC)],
                mb.at[b], s_m).wait()

            @pl.when(cid == 0)
            def _():
                pltpu.make_async_copy(
                    vec_h.at[pl.ds(base, _CHUNK)], vb.at[b], s_v).wait()

        for b in range(_NBUF):
            start_loads(b, b)

        def body(t, carry):
            j0 = t * _NBUF
            for b in range(_NBUF):
                wait_loads(b)
            for b in range(_NBUF):
                pltpu.async_copy(mb.at[b], accm.at[idx.at[b]], s_a, add=True)

                @pl.when(cid == 0)
                def _():
                    pltpu.async_copy(vb.at[b], accv.at[idx.at[b]], s_a,
                                     add=True)

            for b in range(_NBUF):
                pltpu.make_async_copy(mb.at[b], accm.at[idx.at[b]], s_a).wait()

                @pl.when(cid == 0)
                def _():
                    pltpu.make_async_copy(
                        vb.at[b], accv.at[idx.at[b]], s_a).wait()

            @pl.when(t < _NCHUNK_SC // _NBUF - 1)
            def _():
                for b in range(_NBUF):
                    start_loads(j0 + _NBUF + b, b)

            return carry

        lax.fori_loop(0, _NCHUNK_SC // _NBUF, body, 0)
        plsc.subcore_barrier()
        for i, (off, ln) in enumerate(rchunks):
            b = i % _NBUF
            pltpu.sync_copy(accm.at[pl.ds(sid * _RPS + off, ln)],
                            mb.at[b, pl.ds(0, ln)])
            pltpu.sync_copy(mb.at[b, pl.ds(0, ln)],
                            outm_h.at[pl.ds(sid * _RPS + off, ln),
                                      pl.ds(coff, _HC)])

            @pl.when(cid == 0)
            def _():
                pltpu.sync_copy(accv.at[pl.ds(sid * _RPS + off, ln)],
                                vb.at[b, pl.ds(0, ln)])
                pltpu.sync_copy(vb.at[b, pl.ds(0, ln)],
                                outv_h.at[pl.ds(sid * _RPS + off, ln)])

    return k(msg, vec, row, zm, zv)


def _edge_body(hr_ref, hc_ref, zr_ref, zc_ref, ef, w1s, w1r, w1c, w1e, b1,
               w2, b2, ow1, ob1, w2v, b2v, scc, scr, ucc, msg_out, vec_out):
    hr = hr_ref[...]
    zr = zr_ref[...]
    hc = hc_ref[...]
    zc = zc_ref[...]

    prod = _dot(zc, scc[...]) * _dot(zr, scr[...])
    scal = prod[:, :16] + prod[:, 16:32] + prod[:, 32:48]
    nrm = jnp.sqrt(jnp.sum(scal * scal, axis=1, keepdims=True))
    scal = scal / jnp.maximum(nrm, 1e-12)

    acc = (_dot(scal, w1s[...]) + _dot(hr, w1r[...]) + _dot(hc, w1c[...])
           + _dot(ef[...], w1e[...]) + b1[...])
    x = _silu(acc)
    m = _silu(_dot(x, w2[...]) + b2[...])
    msg_out[...] = m

    y = _silu(_dot(m, ow1[...]) + ob1[...])
    vsel = _dot(y, w2v[...]) + b2v[...]
    p2 = _dot(zc, ucc[...]) * vsel
    vec = p2[:, :16] + p2[:, 16:32] + p2[:, 32:48]
    cnt = (lax.broadcasted_iota(jnp.int32, vec.shape, 1) == 9)
    vec_out[...] = vec + cnt.astype(jnp.float32)


def _node_body(am, av, hb, vb, nw1, nb1, nw2, nb2,
               hout, vout):
    tot = am[...]
    vt = av[...]
    cnt = jnp.maximum(vt[:, 9:10], 1.0)
    vout[...] = vt / cnt + vb[...]
    hbv = hb[...]
    z = _silu(_dot(hbv, nw1[:H]) + _dot(tot, nw1[H:]) + nb1[...])
    hout[...] = _dot(z, nw2[...]) + nb2[...] + hbv


_BE = 1280
_BN = 1000


def _tc_edge(hr, hc, zr, zc, ef, *ws):
    grid = E // _BE
    full = lambda a: pl.BlockSpec(a.shape, lambda i: (0,) * a.ndim)
    in_specs = [
        pl.BlockSpec((_BE, H), lambda i: (i, 0)),
        pl.BlockSpec((_BE, H), lambda i: (i, 0)),
        pl.BlockSpec((_BE, 16), lambda i: (i, 0)),
        pl.BlockSpec((_BE, 16), lambda i: (i, 0)),
        pl.BlockSpec((_BE, 4), lambda i: (i, 0)),
    ] + [full(w) for w in ws]
    return pl.pallas_call(
        _edge_body,
        grid=grid,
        in_specs=in_specs,
        out_specs=(pl.BlockSpec((_BE, H), lambda i: (i, 0)),
                   pl.BlockSpec((_BE, 16), lambda i: (i, 0))),
        out_shape=(jax.ShapeDtypeStruct((E, H), jnp.float32),
                   jax.ShapeDtypeStruct((E, 16), jnp.float32)),
        compiler_params=pltpu.CompilerParams(
            dimension_semantics=("arbitrary",)),
    )(hr, hc, zr, zc, ef, *ws)


def _tc_node(accm, accv, h, vb, nw1, nb1, nw2, nb2):
    grid = N // _BN
    full = lambda a: pl.BlockSpec(a.shape, lambda i: (0,) * a.ndim)
    in_specs = [
        pl.BlockSpec((_BN, H), lambda i: (i, 0)),
        pl.BlockSpec((_BN, 16), lambda i: (i, 0)),
        pl.BlockSpec((_BN, H), lambda i: (i, 0)),
        pl.BlockSpec((_BN, 16), lambda i: (i, 0)),
        full(nw1), full(nb1), full(nw2), full(nb2),
    ]
    return pl.pallas_call(
        _node_body,
        grid=grid,
        in_specs=in_specs,
        out_specs=(pl.BlockSpec((_BN, H), lambda i: (i, 0)),
                   pl.BlockSpec((_BN, 16), lambda i: (i, 0))),
        out_shape=(jax.ShapeDtypeStruct((N, H), jnp.float32),
                   jax.ShapeDtypeStruct((N, 16), jnp.float32)),
        compiler_params=pltpu.CompilerParams(
            dimension_semantics=("arbitrary",)),
    )(accm, accv, h, vb, nw1, nb1, nw2, nb2)


def kernel(vectors, h, edge_index, edge_fea,
           in_w1, in_b1, in_w2, in_b2,
           out_w1, out_b1, out_w2, out_b2,
           node_w1, node_b1, node_w2, node_b2):
    row = edge_index[0].astype(jnp.int32)
    col = edge_index[1].astype(jnp.int32)
    vec9 = vectors.reshape(N, 9)
    vb16 = jnp.concatenate([vec9, jnp.zeros((N, 7), jnp.float32)], axis=1)

    hr, hc = _sc_gather_h(h, row.reshape(E // _CHUNK, _CHUNK),
                          col.reshape(E // _CHUNK, _CHUNK))
    zr, zc = _sc_gather_z(vb16, row, col)

    scc = jnp.asarray(_SEL_SC_C)
    scr = jnp.asarray(_SEL_SC_R)
    ucc = jnp.asarray(_SEL_U_C)
    w2v = out_w2 @ jnp.asarray(_SEL_U_V)
    b2v = out_b2.reshape(1, 9) @ jnp.asarray(_SEL_U_V)
    w1s = jnp.concatenate([in_w1[:9], jnp.zeros((7, H), jnp.float32)], axis=0)
    ws = (w1s, in_w1[9:9 + H], in_w1[9 + H:9 + 2 * H], in_w1[9 + 2 * H:],
          in_b1.reshape(1, H), in_w2, in_b2.reshape(1, H),
          out_w1, out_b1.reshape(1, H), w2v, b2v, scc, scr, ucc)

    msg, vec = _tc_edge(hr, hc, zr, zc, edge_fea, *ws)

    zm = jnp.zeros((_CHUNK, _HC), jnp.float32)
    zv = jnp.zeros((_CHUNK, 16), jnp.float32)
    accm, accv = _sc_scatter(msg, vec, row, zm, zv)

    h_new, vout16 = _tc_node(accm, accv, h, vb16,
                             node_w1, node_b1.reshape(1, H),
                             node_w2, node_b2.reshape(1, H))
    vout = vout16[:, :9].reshape(N, 3, 3)
    return (vout, h_new)
